# baseline (speedup 1.0000x reference)
import functools

import jax
import jax.numpy as jnp
from jax import lax
from jax.experimental import pallas as pl
from jax.experimental.pallas import tpu as pltpu
from jax.experimental.pallas import tpu_sc as plsc

NUM_TOKENS = 100000
MAX_LENGTH = 2048
EMB_SIZE = 768
BATCH = 4
SEQ_LEN = 2048

L = 16
NW = 32
S_PER_W = SEQ_LEN // NW
CHUNK = 32
N_CHUNKS = BATCH * S_PER_W // CHUNK
H = S_PER_W // CHUNK
VCH = EMB_SIZE // L


def _tok_pos_kernel(ids_hbm, table_hbm, pos_hbm, out_hbm,
                    idx_v, pos_v, rows0, rows1,
                    isem, psem, gsem0, gsem1, osem0, osem1):
    wid = lax.axis_index("s") * 2 + lax.axis_index("c")
    base = wid * S_PER_W

    rows = (rows0, rows1)
    gsem = (gsem0, gsem1)
    osem = (osem0, osem1)

    id_cps = [
        pltpu.async_copy(ids_hbm.at[b, pl.ds(base, S_PER_W)], idx_v.at[b], isem)
        for b in range(BATCH)
    ]
    pos_cp = pltpu.async_copy(pos_hbm.at[pl.ds(base, S_PER_W)], pos_v, psem)
    for cp in id_cps:
        cp.wait()

    cps = {}

    def start(k):
        b, h = divmod(k, H)
        buf = k & 1
        idx = idx_v.at[b, pl.ds(h * CHUNK, CHUNK)]
        cps[k] = pltpu.async_copy(table_hbm.at[idx], rows[buf], gsem[buf])

    def finish(k):
        b, h = divmod(k, H)
        buf = k & 1
        cps[k].wait()

        def add_row(r, carry):
            for j in range(VCH):
                sl = pl.ds(j * L, L)
                plsc.addupdate(rows[buf].at[r, sl], pos_v[h * CHUNK + r, sl])
            return carry

        lax.fori_loop(0, CHUNK, add_row, None)
        cps[N_CHUNKS + k] = pltpu.async_copy(
            rows[buf], out_hbm.at[b, pl.ds(base + h * CHUNK, CHUNK), :], osem[buf])

    start(0)
    pos_cp.wait()
    for k in range(1, N_CHUNKS):
        if k >= 2:
            cps[N_CHUNKS + k - 2].wait()
        start(k)
        finish(k - 1)
    cps[2 * N_CHUNKS - 2].wait()
    finish(N_CHUNKS - 1)
    cps[2 * N_CHUNKS - 1].wait()


@jax.jit
def _tok_pos(token_ids, token_table, pos_table):
    mesh = plsc.VectorSubcoreMesh(core_axis_name="c", subcore_axis_name="s")
    run = functools.partial(
        pl.kernel,
        mesh=mesh,
        out_type=jax.ShapeDtypeStruct((BATCH, SEQ_LEN, EMB_SIZE), jnp.float32),
        scratch_types=[
            pltpu.VMEM((BATCH, S_PER_W), jnp.int32),
            pltpu.VMEM((S_PER_W, EMB_SIZE), jnp.float32),
            pltpu.VMEM((CHUNK, EMB_SIZE), jnp.float32),
            pltpu.VMEM((CHUNK, EMB_SIZE), jnp.float32),
            pltpu.SemaphoreType.DMA,
            pltpu.SemaphoreType.DMA,
            pltpu.SemaphoreType.DMA,
            pltpu.SemaphoreType.DMA,
            pltpu.SemaphoreType.DMA,
            pltpu.SemaphoreType.DMA,
        ],
    )(_tok_pos_kernel)
    return run(token_ids, token_table, pos_table)


def kernel(token_ids, token_table, pos_table):
    return _tok_pos(token_ids.astype(jnp.int32), token_table, pos_table)


# speedup vs baseline: 1.0757x; 1.0757x over previous
import functools

import jax
import jax.numpy as jnp
from jax import lax
from jax.experimental import pallas as pl
from jax.experimental.pallas import tpu as pltpu
from jax.experimental.pallas import tpu_sc as plsc

NUM_TOKENS = 100000
MAX_LENGTH = 2048
EMB_SIZE = 768
BATCH = 4
SEQ_LEN = 2048

L = 16
NW = 32
NS = 16
S_PER_W = SEQ_LEN // NW
CHUNK = 16
NH = S_PER_W // CHUNK
VCH = EMB_SIZE // L
SROWS = 8
NSUB = BATCH * NH * (CHUNK // SROWS)


def _tok_pos_kernel(ids_hbm, table_hbm, pos_hbm, out_hbm,
                    idx_v,
                    r00, r01, r02, r03, r10, r11, r12, r13,
                    posb, stage0, stage1,
                    isem, psem, xsem0, xsem1, lsem0, lsem1,
                    gs00, gs01, gs02, gs03, gs10, gs11, gs12, gs13):
    sid = lax.axis_index("s")
    cid = lax.axis_index("c")
    wid = sid * 2 + cid
    base = wid * S_PER_W
    sbase = sid * SROWS

    rows = ((r00, r01, r02, r03), (r10, r11, r12, r13))
    stage = (stage0, stage1)
    xsem = (xsem0, xsem1)
    lsem = (lsem0, lsem1)
    gsem = ((gs00, gs01, gs02, gs03), (gs10, gs11, gs12, gs13))

    id_cps = [
        pltpu.async_copy(ids_hbm.at[b, pl.ds(base, S_PER_W)], idx_v.at[b], isem)
        for b in range(BATCH)
    ]
    for cp in id_cps:
        cp.wait()

    gcps = {}
    pcps = {}
    xcps = {}
    lcps = {}

    def load_pos(h):
        pcps[h] = pltpu.async_copy(
            pos_hbm.at[pl.ds(base + h * CHUNK, CHUNK)], posb, psem)

    def start(h):
        buf = h & 1
        for b in range(BATCH):
            idx = idx_v.at[b, pl.ds(h * CHUNK, CHUNK)]
            gcps[(h, b)] = pltpu.async_copy(
                table_hbm.at[idx], rows[buf][b], gsem[buf][b])

    def drain_out(s):
        if 0 <= s < NSUB:
            xcps[s].wait()
            hb, half = divmod(s, CHUNK // SROWS)
            h, b = divmod(hb, BATCH)
            lcps[s] = pltpu.async_copy(
                stage[s & 1].at[pl.ds(sbase, SROWS)],
                out_hbm.at[b, pl.ds(base + h * CHUNK + half * SROWS, SROWS), :],
                lsem[s & 1])

    def finish(h):
        buf = h & 1
        pcps[h].wait()
        for b in range(BATCH):
            gcps[(h, b)].wait()

        def add_row(r, carry):
            for j in range(VCH):
                sl = pl.ds(j * L, L)
                p = posb[r, sl]
                for b in range(BATCH):
                    plsc.addupdate(rows[buf][b].at[r, sl], p)
            return carry

        lax.fori_loop(0, CHUNK, add_row, None)
        if h + 1 < NH:
            load_pos(h + 1)
        for b in range(BATCH):
            for half in range(CHUNK // SROWS):
                s = (h * BATCH + b) * (CHUNK // SROWS) + half
                if s >= 2:
                    lcps[s - 2].wait()
                xcps[s] = pltpu.async_copy(
                    rows[buf][b].at[pl.ds(half * SROWS, SROWS), :],
                    stage[s & 1].at[pl.ds(sbase, SROWS)],
                    xsem[s & 1])
                drain_out(s - 1)

    load_pos(0)
    start(0)
    for h in range(1, NH):
        start(h)
        finish(h - 1)
    finish(NH - 1)
    drain_out(NSUB - 1)
    lcps[NSUB - 2].wait()
    lcps[NSUB - 1].wait()


@jax.jit
def _tok_pos(token_ids, token_table, pos_table):
    mesh = plsc.VectorSubcoreMesh(core_axis_name="c", subcore_axis_name="s")
    rows_t = [pltpu.VMEM((CHUNK, EMB_SIZE), jnp.float32) for _ in range(8)]
    pos_t = [pltpu.VMEM((CHUNK, EMB_SIZE), jnp.float32)]
    stage_t = [pltpu.VMEM_SHARED((NS * SROWS, EMB_SIZE), jnp.float32)
               for _ in range(2)]
    sems = [pltpu.SemaphoreType.DMA for _ in range(14)]
    run = functools.partial(
        pl.kernel,
        mesh=mesh,
        out_type=jax.ShapeDtypeStruct((BATCH, SEQ_LEN, EMB_SIZE), jnp.float32),
        scratch_types=[pltpu.VMEM((BATCH, S_PER_W), jnp.int32)]
        + rows_t + pos_t + stage_t + sems,
    )(_tok_pos_kernel)
    return run(token_ids, token_table, pos_table)


def kernel(token_ids, token_table, pos_table):
    return _tok_pos(token_ids.astype(jnp.int32), token_table, pos_table)
